# Initial kernel scaffold; baseline (speedup 1.0000x reference)
import jax
import jax.numpy as jnp
from jax.experimental import pallas as pl


def kernel(token_ids, table, prompt):
    raise NotImplementedError("write your pallas kernel here")


# speedup vs baseline: 1.5849x; 1.5849x over previous
import functools

import jax
import jax.numpy as jnp
from jax import lax
from jax.experimental import pallas as pl
from jax.experimental.pallas import tpu as pltpu
from jax.experimental.pallas import tpu_sc as plsc

VOCAB = 100000
DIM = 128
BATCH = 16384

_info = plsc.get_sparse_core_info()
_NC = _info.num_cores
_NS = _info.num_subcores
_L = _info.num_lanes
_NW = _NC * _NS
_BPW = BATCH // _NW
_PCH = _BPW // 2


def _body(idx_hbm, table_hbm, prompt_hbm, out_hbm, idx_v, rows_v, prompt_v,
          pbuf_v, gsem):
    wid = lax.axis_index("s") * _NC + lax.axis_index("c")
    base = wid * _BPW

    pltpu.sync_copy(idx_hbm.at[pl.ds(base, _BPW)], idx_v)
    gather = pltpu.async_copy(table_hbm.at[idx_v], rows_v, gsem)

    pltpu.sync_copy(prompt_hbm, pbuf_v)
    pvecs = [pbuf_v[0, pl.ds(j * _L, _L)] for j in range(DIM // _L)]

    def fill_row(i, carry):
        for j in range(DIM // _L):
            prompt_v[i, pl.ds(j * _L, _L)] = pvecs[j]
        return carry

    lax.fori_loop(0, _PCH, fill_row, 0)

    pltpu.sync_copy(prompt_v, out_hbm.at[pl.ds(base, _PCH), pl.ds(0, DIM)])
    pltpu.sync_copy(prompt_v,
                    out_hbm.at[pl.ds(base + _PCH, _PCH), pl.ds(0, DIM)])

    gather.wait()
    pltpu.sync_copy(rows_v, out_hbm.at[pl.ds(base, _BPW), pl.ds(DIM, DIM)])


@jax.jit
def _run(token_ids, table, prompt):
    mesh = plsc.VectorSubcoreMesh(core_axis_name="c", subcore_axis_name="s")
    f = functools.partial(
        pl.kernel,
        mesh=mesh,
        out_type=jax.ShapeDtypeStruct((BATCH, 2 * DIM), jnp.float32),
        scratch_types=[
            pltpu.VMEM((_BPW,), jnp.int32),
            pltpu.VMEM((_BPW, DIM), jnp.float32),
            pltpu.VMEM((_PCH, DIM), jnp.float32),
            pltpu.VMEM((1, DIM), jnp.float32),
            pltpu.SemaphoreType.DMA,
        ],
    )(_body)
    return f(token_ids, table, prompt)


def kernel(token_ids, table, prompt):
    return _run(token_ids.astype(jnp.int32), table, prompt)
